# baseline (speedup 1.0000x reference)
import functools

import jax
import jax.numpy as jnp
from jax import lax
from jax.experimental import pallas as pl
from jax.experimental.pallas import tpu as pltpu
from jax.experimental.pallas import tpu_sc as plsc

N = 10000
E = 320000
D = 128

NC = 2
NS = 16
N_ACC = 10240
ROWS_PT = N_ACC // NS
EPT = E // (NC * NS)
K = 80
NBLK = EPT // K
NBUF = 4
PAR = 2
NMAIN = 15

_sc_mesh = plsc.VectorSubcoreMesh(core_axis_name="c", subcore_axis_name="s")


@functools.partial(
    pl.kernel,
    out_type=pltpu.HBM((2 * N, D), jnp.float32),
    mesh=_sc_mesh,
    scratch_types=[
        pltpu.VMEM((NBUF * PAR, K), jnp.int32),
        pltpu.VMEM((NBUF * PAR, K), jnp.int32),
        pltpu.VMEM((K, D), jnp.float32),
        pltpu.VMEM((K, D), jnp.float32),
        pltpu.VMEM((K, D), jnp.float32),
        pltpu.VMEM((K, D), jnp.float32),
        pltpu.VMEM_SHARED((N_ACC, D), jnp.float32),
        pltpu.SemaphoreType.DMA,
        pltpu.SemaphoreType.DMA,
        pltpu.SemaphoreType.DMA,
        pltpu.SemaphoreType.DMA,
        pltpu.SemaphoreType.DMA,
        pltpu.SemaphoreType.DMA,
        pltpu.SemaphoreType.DMA,
        pltpu.SemaphoreType.DMA,
        pltpu.SemaphoreType.DMA,
        pltpu.SemaphoreType.DMA,
        pltpu.SemaphoreType.DMA,
        pltpu.SemaphoreType.DMA,
        pltpu.SemaphoreType.DMA,
        pltpu.SemaphoreType.DMA,
        pltpu.SemaphoreType.DMA,
        pltpu.SemaphoreType.DMA,
    ],
)
def _sc_aggregate(x_hbm, edge_hbm, zero_hbm, out_hbm,
                  srci_v, dsti_v, rows0, rows1, rows2, rows3, acc_sh,
                  g0, g1, g2, g3, sc0, sc1, sc2, sc3,
                  i0, i1, i2, i3, i4, i5, i6, i7):
    c = lax.axis_index("c")
    s = lax.axis_index("s")
    row0 = s * ROWS_PT
    e0 = (c * NS + s) * EPT
    rows = (rows0, rows1, rows2, rows3)
    gsem = (g0, g1, g2, g3)
    ssem = (sc0, sc1, sc2, sc3)
    isem = (i0, i1, i2, i3, i4, i5, i6, i7)

    def issue_idx(i, j):
        off = e0 + i * K
        pltpu.async_copy(edge_hbm.at[pl.ds(off, K)], srci_v.at[j], isem[j])
        pltpu.async_copy(edge_hbm.at[pl.ds(E + off, K)], dsti_v.at[j], isem[j])

    def wait_idx(i, j):
        off = e0 + i * K
        pltpu.make_async_copy(
            edge_hbm.at[pl.ds(off, K)], srci_v.at[j], isem[j]).wait()
        pltpu.make_async_copy(
            edge_hbm.at[pl.ds(E + off, K)], dsti_v.at[j], isem[j]).wait()

    def issue_gather(b, j):
        pltpu.async_copy(x_hbm.at[srci_v.at[j]], rows[b], gsem[b])

    def wait_gather(b, j):
        pltpu.make_async_copy(
            x_hbm.at[srci_v.at[j]], rows[b], gsem[b]).wait()

    def issue_scatter(b, j):
        pltpu.async_copy(rows[b], acc_sh.at[dsti_v.at[j]], ssem[b], add=True)

    def wait_scatter(b, j):
        pltpu.make_async_copy(
            rows[b], acc_sh.at[dsti_v.at[j]], ssem[b]).wait()

    for b in range(NBUF):
        issue_idx(b, b * PAR)
        issue_idx(b + NBUF, b * PAR + 1)
    pltpu.sync_copy(zero_hbm, acc_sh.at[pl.ds(row0, ROWS_PT)])
    for b in range(NBUF):
        wait_idx(b, b * PAR)
        issue_gather(b, b * PAR)
    plsc.subcore_barrier()

    def visit(i, b, p):
        j = b * PAR + p
        pb = (b - 1) % NBUF
        pp = p if b > 0 else 1 - p
        jp = pb * PAR + pp
        jg = pb * PAR + (1 - pp)
        wait_gather(b, j)
        issue_scatter(b, j)

        @pl.when(i >= 1)
        def _():
            wait_scatter(pb, jp)

            @pl.when(i + NBUF - 1 < NBLK)
            def _():
                wait_idx(i + NBUF - 1, jg)
                issue_gather(pb, jg)

            @pl.when(i - 1 + 2 * NBUF < NBLK)
            def _():
                issue_idx(i - 1 + 2 * NBUF, jp)

    def group(g, carry):
        for gg in range(PAR):
            for b in range(NBUF):
                visit((g * PAR + gg) * NBUF + b, b, gg)
        return carry

    lax.fori_loop(0, NMAIN, group, 0)

    for i in range(NMAIN * PAR * NBUF, NBLK):
        visit(i, i % NBUF, (i // NBUF) % PAR)

    b_last = (NBLK - 1) % NBUF
    wait_scatter(b_last, b_last * PAR + ((NBLK - 1) // NBUF) % PAR)

    plsc.subcore_barrier()

    @pl.when(s < NS - 1)
    def _():
        pltpu.sync_copy(acc_sh.at[pl.ds(row0, ROWS_PT)],
                        out_hbm.at[pl.ds(c * N + row0, ROWS_PT)])

    @pl.when(s == NS - 1)
    def _():
        tail = N - (NS - 1) * ROWS_PT
        pltpu.sync_copy(acc_sh.at[pl.ds(row0, tail)],
                        out_hbm.at[pl.ds(c * N + row0, tail)])


_TC_R = 2000


def _tc_matmul_body(a0_ref, a1_ref, w_ref, b_ref, out_ref):
    out_ref[...] = (
        jnp.dot(a0_ref[...] + a1_ref[...], w_ref[...],
                preferred_element_type=jnp.float32)
        + b_ref[...]
    )


def _tc_matmul(agg, W, b):
    b2 = b.reshape(1, D)
    return pl.pallas_call(
        _tc_matmul_body,
        grid=(N // _TC_R,),
        in_specs=[
            pl.BlockSpec((_TC_R, D), lambda i: (i, 0)),
            pl.BlockSpec((_TC_R, D), lambda i: (i + N // _TC_R, 0)),
            pl.BlockSpec((D, D), lambda i: (0, 0)),
            pl.BlockSpec((1, D), lambda i: (0, 0)),
        ],
        out_specs=pl.BlockSpec((_TC_R, D), lambda i: (i, 0)),
        out_shape=jax.ShapeDtypeStruct((N, D), jnp.float32),
    )(agg, agg, W, b2)


@jax.jit
def kernel(x, edge_index, W, b):
    ei = edge_index.astype(jnp.int32).reshape(2 * E)
    zeros = jnp.zeros((ROWS_PT, D), jnp.float32)
    agg = _sc_aggregate(x, ei, zeros)
    return _tc_matmul(agg, W, b)


# speedup vs baseline: 1.0771x; 1.0771x over previous
import functools

import jax
import jax.numpy as jnp
from jax import lax
from jax.experimental import pallas as pl
from jax.experimental.pallas import tpu as pltpu
from jax.experimental.pallas import tpu_sc as plsc

N = 10000
E = 320000
D = 128

NC = 2
NS = 16
N_ACC = 10240
ROWS_PT = N_ACC // NS
EPT = E // (NC * NS)
K = 80
NBLK = EPT // K
NBUF = 4
PAR = 2
NMAIN = 15

_sc_mesh = plsc.VectorSubcoreMesh(core_axis_name="c", subcore_axis_name="s")


@functools.partial(
    pl.kernel,
    out_type=pltpu.HBM((2 * N, D), jnp.float32),
    mesh=_sc_mesh,
    scratch_types=[
        pltpu.VMEM((NBUF * PAR, K), jnp.int32),
        pltpu.VMEM((NBUF * PAR, K), jnp.int32),
        pltpu.VMEM((K, D), jnp.float32),
        pltpu.VMEM((K, D), jnp.float32),
        pltpu.VMEM((K, D), jnp.float32),
        pltpu.VMEM((K, D), jnp.float32),
        pltpu.VMEM_SHARED((N_ACC, D), jnp.float32),
        pltpu.SemaphoreType.DMA,
        pltpu.SemaphoreType.DMA,
        pltpu.SemaphoreType.DMA,
        pltpu.SemaphoreType.DMA,
        pltpu.SemaphoreType.DMA,
        pltpu.SemaphoreType.DMA,
        pltpu.SemaphoreType.DMA,
        pltpu.SemaphoreType.DMA,
        pltpu.SemaphoreType.DMA,
        pltpu.SemaphoreType.DMA,
        pltpu.SemaphoreType.DMA,
        pltpu.SemaphoreType.DMA,
    ],
)
def _sc_aggregate(x_hbm, edge_hbm, zero_hbm, out_hbm,
                  srci_v, dsti_v, rows0, rows1, rows2, rows3, acc_sh,
                  g0, g1, g2, g3,
                  i0, i1, i2, i3, i4, i5, i6, i7):
    c = lax.axis_index("c")
    s = lax.axis_index("s")
    row0 = s * ROWS_PT
    e0 = (c * NS + s) * EPT
    rows = (rows0, rows1, rows2, rows3)
    gsem = (g0, g1, g2, g3)
    isem = (i0, i1, i2, i3, i4, i5, i6, i7)

    def issue_idx(i, j):
        off = e0 + i * K
        pltpu.async_copy(edge_hbm.at[pl.ds(off, K)], srci_v.at[j], isem[j])
        pltpu.async_copy(edge_hbm.at[pl.ds(E + off, K)], dsti_v.at[j], isem[j])

    def wait_idx(i, j):
        off = e0 + i * K
        pltpu.make_async_copy(
            edge_hbm.at[pl.ds(off, K)], srci_v.at[j], isem[j]).wait()
        pltpu.make_async_copy(
            edge_hbm.at[pl.ds(E + off, K)], dsti_v.at[j], isem[j]).wait()

    def issue_gather(b, j):
        pltpu.async_copy(x_hbm.at[srci_v.at[j]], rows[b], gsem[b])

    def wait_gather(b, j):
        pltpu.make_async_copy(
            x_hbm.at[srci_v.at[j]], rows[b], gsem[b]).wait()

    for b in range(NBUF):
        issue_idx(b, b * PAR)
        issue_idx(b + NBUF, b * PAR + 1)
    pltpu.sync_copy(zero_hbm, acc_sh.at[pl.ds(row0, ROWS_PT)])
    for b in range(NBUF):
        wait_idx(b, b * PAR)
        issue_gather(b, b * PAR)
    plsc.subcore_barrier()

    def visit(i, b, p):
        j = b * PAR + p
        jn = b * PAR + (1 - p)
        wait_gather(b, j)
        pltpu.sync_copy(rows[b], acc_sh.at[dsti_v.at[j]], add=True)

        @pl.when(i + NBUF < NBLK)
        def _():
            wait_idx(i + NBUF, jn)
            issue_gather(b, jn)

        @pl.when(i + 2 * NBUF < NBLK)
        def _():
            issue_idx(i + 2 * NBUF, j)

    def group(g, carry):
        for gg in range(PAR):
            for b in range(NBUF):
                visit((g * PAR + gg) * NBUF + b, b, gg)
        return carry

    lax.fori_loop(0, NMAIN, group, 0)

    for i in range(NMAIN * PAR * NBUF, NBLK):
        visit(i, i % NBUF, (i // NBUF) % PAR)

    plsc.subcore_barrier()

    @pl.when(s < NS - 1)
    def _():
        pltpu.sync_copy(acc_sh.at[pl.ds(row0, ROWS_PT)],
                        out_hbm.at[pl.ds(c * N + row0, ROWS_PT)])

    @pl.when(s == NS - 1)
    def _():
        tail = N - (NS - 1) * ROWS_PT
        pltpu.sync_copy(acc_sh.at[pl.ds(row0, tail)],
                        out_hbm.at[pl.ds(c * N + row0, tail)])


_TC_R = 2000


def _tc_matmul_body(a0_ref, a1_ref, w_ref, b_ref, out_ref):
    out_ref[...] = (
        jnp.dot(a0_ref[...] + a1_ref[...], w_ref[...],
                preferred_element_type=jnp.float32)
        + b_ref[...]
    )


def _tc_matmul(agg, W, b):
    b2 = b.reshape(1, D)
    return pl.pallas_call(
        _tc_matmul_body,
        grid=(N // _TC_R,),
        in_specs=[
            pl.BlockSpec((_TC_R, D), lambda i: (i, 0)),
            pl.BlockSpec((_TC_R, D), lambda i: (i + N // _TC_R, 0)),
            pl.BlockSpec((D, D), lambda i: (0, 0)),
            pl.BlockSpec((1, D), lambda i: (0, 0)),
        ],
        out_specs=pl.BlockSpec((_TC_R, D), lambda i: (i, 0)),
        out_shape=jax.ShapeDtypeStruct((N, D), jnp.float32),
    )(agg, agg, W, b2)


@jax.jit
def kernel(x, edge_index, W, b):
    ei = edge_index.astype(jnp.int32).reshape(2 * E)
    zeros = jnp.zeros((ROWS_PT, D), jnp.float32)
    agg = _sc_aggregate(x, ei, zeros)
    return _tc_matmul(agg, W, b)
